# baseline (speedup 1.0000x reference)
import functools

import jax
import jax.numpy as jnp
from jax import lax
from jax.experimental import pallas as pl
from jax.experimental.pallas import tpu as pltpu
from jax.experimental.pallas import tpu_sc as plsc

D = 192
K = 1024
EPS = 1e-5

ROWS_BLK = 512


def _conv(x, Wt, b, stride, pad):
    y = lax.conv_general_dilated(x, Wt, (stride, stride), ((pad, pad), (pad, pad)),
                                 dimension_numbers=('NCHW', 'OIHW', 'NCHW'))
    return y + b[None, :, None, None]


def _deconv(x, Wt, b, stride=2, pad=1, k=4):
    Wf = jnp.flip(Wt, (2, 3)).transpose(1, 0, 2, 3)
    q = k - 1 - pad
    y = lax.conv_general_dilated(x, Wf, (1, 1), ((q, q), (q, q)), lhs_dilation=(stride, stride),
                                 dimension_numbers=('NCHW', 'OIHW', 'NCHW'))
    return y + b[None, :, None, None]


def _bn(x, g, b):
    return g[None, :, None, None] * x / jnp.sqrt(1.0 + EPS) + b[None, :, None, None]


def _resblock(x, i, W3, b3, g1, be1, W1, b1, g2, be2):
    h = jax.nn.relu(x)
    h = _conv(h, W3[i], b3[i], 1, 1)
    h = _bn(h, g1[i], be1[i])
    h = jax.nn.relu(h)
    h = _conv(h, W1[i], b1[i], 1, 0)
    h = _bn(h, g2[i], be2[i])
    return x + h


K_CHUNK = 128


def _vq_block_kernel(z_ref, csq_ref, cb_ref, idx_ref):
    flat = jnp.transpose(z_ref[0], (1, 0))
    rows = flat.shape[0]
    sum1 = jnp.sum(flat * flat, axis=1, keepdims=True)
    csq = csq_ref[...]
    cb = cb_ref[...]
    best_d = None
    best_i = None
    for c0 in range(0, K, K_CHUNK):
        cbc = cb[c0:c0 + K_CHUNK]
        m = jax.lax.dot_general(flat, cbc, (((1,), (1,)), ((), ())),
                                preferred_element_type=jnp.float32)
        d = (sum1 - 2.0 * m) + csq[:, c0:c0 + K_CHUNK]
        dmin = jnp.min(d, axis=1, keepdims=True)
        iota = c0 + jax.lax.broadcasted_iota(jnp.int32, (rows, K_CHUNK), 1)
        imin = jnp.min(jnp.where(d == dmin, iota, K), axis=1, keepdims=True)
        if best_d is None:
            best_d, best_i = dmin, imin
        else:
            take_new = dmin < best_d
            best_i = jnp.where(take_new, imin, best_i)
            best_d = jnp.where(take_new, dmin, best_d)
    idx_ref[...] = jnp.broadcast_to(best_i[:, 0][None, None, :], idx_ref.shape)


def _vq_indices(z_e_x, codebook):
    b, _, hh, ww = z_e_x.shape
    hw = hh * ww
    z3 = z_e_x.reshape(b, D, hw)
    csq = jnp.sum(codebook ** 2, axis=1)[None, :]
    idx = pl.pallas_call(
        _vq_block_kernel,
        grid=(b,),
        in_specs=[
            pl.BlockSpec((1, D, hw), lambda i: (i, 0, 0)),
            pl.BlockSpec((1, K), lambda i: (0, 0)),
            pl.BlockSpec((K, D), lambda i: (0, 0)),
        ],
        out_specs=pl.BlockSpec((1, 8, hw), lambda i: (i, 0, 0)),
        out_shape=jax.ShapeDtypeStruct((b, 8, hw), jnp.int32),
    )(z3, csq, codebook)
    return idx[:, 0, :].reshape(b * hw)


DPAD = 256


def _sc_gather(codebook, idx):
    n = idx.shape[0]
    table = jnp.pad(codebook, ((0, 0), (0, DPAD - D)))
    info = plsc.get_sparse_core_info()
    nw = info.num_cores * info.num_subcores
    b_per_w = n // nw
    n_chunks = 2
    chunk = b_per_w // n_chunks
    mesh = plsc.VectorSubcoreMesh(core_axis_name="c", subcore_axis_name="s")

    @functools.partial(
        pl.kernel, mesh=mesh,
        out_type=jax.ShapeDtypeStruct((n, DPAD), jnp.float32),
        scratch_types=[
            pltpu.VMEM((chunk,), jnp.int32),
            pltpu.VMEM((chunk, DPAD), jnp.float32),
            pltpu.SemaphoreType.DMA,
        ],
    )
    def gather_kernel(table_hbm, idx_hbm, out_hbm, idx_v, rows_v, sem):
        wid = lax.axis_index("s") * info.num_cores + lax.axis_index("c")
        for j in range(n_chunks):
            base = wid * b_per_w + j * chunk
            pltpu.sync_copy(idx_hbm.at[pl.ds(base, chunk)], idx_v)
            pltpu.async_copy(table_hbm.at[idx_v], rows_v, sem).wait()
            pltpu.sync_copy(rows_v, out_hbm.at[pl.ds(base, chunk)])

    return gather_kernel(table, idx)[:, :D]


def kernel(x, conv1_W, conv1_b, bn1_g, bn1_b, conv2_W, conv2_b,
           res_W3, res_b3, res_g1, res_be1, res_W1, res_b1, res_g2, res_be2,
           deconv1_W, deconv1_b, bn2_g, bn2_b, deconv2_W, deconv2_b, codebook):
    h = _conv(x, conv1_W, conv1_b, 2, 1)
    h = _bn(h, bn1_g, bn1_b)
    h = jax.nn.relu(h)
    h = _conv(h, conv2_W, conv2_b, 2, 1)
    h = _resblock(h, 0, res_W3, res_b3, res_g1, res_be1, res_W1, res_b1, res_g2, res_be2)
    z_e_x = _resblock(h, 1, res_W3, res_b3, res_g1, res_be1, res_W1, res_b1, res_g2, res_be2)
    idx = _vq_indices(z_e_x, codebook)
    zq_flat = _sc_gather(codebook, idx)
    bsz, _, hh, ww = z_e_x.shape
    z_q_perm = zq_flat.reshape(bsz, hh, ww, D)
    z_q_x = z_q_perm.transpose(0, 3, 1, 2)
    z_q_x_st = z_e_x + lax.stop_gradient(z_q_x - z_e_x)
    h = _resblock(z_q_x_st, 2, res_W3, res_b3, res_g1, res_be1, res_W1, res_b1, res_g2, res_be2)
    h = _resblock(h, 3, res_W3, res_b3, res_g1, res_be1, res_W1, res_b1, res_g2, res_be2)
    h = jax.nn.relu(h)
    h = _deconv(h, deconv1_W, deconv1_b, 2, 1, 4)
    h = _bn(h, bn2_g, bn2_b)
    h = jax.nn.relu(h)
    h = _deconv(h, deconv2_W, deconv2_b, 2, 1, 4)
    x_tilde = jnp.tanh(h)
    return (x_tilde, z_e_x, z_q_x)


# speedup vs baseline: 1.0623x; 1.0623x over previous
import functools

import jax
import jax.numpy as jnp
from jax import lax
from jax.experimental import pallas as pl
from jax.experimental.pallas import tpu as pltpu
from jax.experimental.pallas import tpu_sc as plsc

D = 192
K = 1024
EPS = 1e-5

ROWS_BLK = 896


def _conv(x, Wt, b, stride, pad):
    y = lax.conv_general_dilated(x, Wt, (stride, stride), ((pad, pad), (pad, pad)),
                                 dimension_numbers=('NCHW', 'OIHW', 'NCHW'))
    return y + b[None, :, None, None]


def _deconv(x, Wt, b, stride=2, pad=1, k=4):
    Wf = jnp.flip(Wt, (2, 3)).transpose(1, 0, 2, 3)
    q = k - 1 - pad
    y = lax.conv_general_dilated(x, Wf, (1, 1), ((q, q), (q, q)), lhs_dilation=(stride, stride),
                                 dimension_numbers=('NCHW', 'OIHW', 'NCHW'))
    return y + b[None, :, None, None]


def _bn(x, g, b):
    return g[None, :, None, None] * x / jnp.sqrt(1.0 + EPS) + b[None, :, None, None]


def _resblock(x, i, W3, b3, g1, be1, W1, b1, g2, be2):
    h = jax.nn.relu(x)
    h = _conv(h, W3[i], b3[i], 1, 1)
    h = _bn(h, g1[i], be1[i])
    h = jax.nn.relu(h)
    h = _conv(h, W1[i], b1[i], 1, 0)
    h = _bn(h, g2[i], be2[i])
    return x + h


def _vq_block_kernel(flat_ref, csq_ref, cb_ref, idx_ref):
    flat = flat_ref[...]
    m = jax.lax.dot_general(flat, cb_ref[...], (((1,), (1,)), ((), ())),
                            preferred_element_type=jnp.float32)
    sum1 = jnp.sum(flat * flat, axis=1, keepdims=True)
    d = (sum1 - 2.0 * m) + csq_ref[...]
    dmin = jnp.min(d, axis=1, keepdims=True)
    iota = jax.lax.broadcasted_iota(jnp.int32, (ROWS_BLK, K), 1)
    idx_ref[...] = jnp.min(jnp.where(d == dmin, iota, K), axis=1, keepdims=True)


def _vq_indices(flat, codebook):
    n = flat.shape[0]
    csq = jnp.sum(codebook ** 2, axis=1)[None, :]
    grid = n // ROWS_BLK
    idx = pl.pallas_call(
        _vq_block_kernel,
        grid=(grid,),
        in_specs=[
            pl.BlockSpec((ROWS_BLK, D), lambda i: (i, 0)),
            pl.BlockSpec((1, K), lambda i: (0, 0)),
            pl.BlockSpec((K, D), lambda i: (0, 0)),
        ],
        out_specs=pl.BlockSpec((ROWS_BLK, 1), lambda i: (i, 0)),
        out_shape=jax.ShapeDtypeStruct((n, 1), jnp.int32),
    )(flat, csq, codebook)
    return idx.reshape(n)


DPAD = 256


def _sc_gather(codebook, idx):
    n = idx.shape[0]
    table = jnp.pad(codebook, ((0, 0), (0, DPAD - D)))
    info = plsc.get_sparse_core_info()
    nw = info.num_cores * info.num_subcores
    b_per_w = n // nw
    n_chunks = 2
    chunk = b_per_w // n_chunks
    mesh = plsc.VectorSubcoreMesh(core_axis_name="c", subcore_axis_name="s")

    @functools.partial(
        pl.kernel, mesh=mesh,
        out_type=jax.ShapeDtypeStruct((n, DPAD), jnp.float32),
        scratch_types=[
            pltpu.VMEM((chunk,), jnp.int32),
            pltpu.VMEM((chunk, DPAD), jnp.float32),
            pltpu.SemaphoreType.DMA,
        ],
    )
    def gather_kernel(table_hbm, idx_hbm, out_hbm, idx_v, rows_v, sem):
        wid = lax.axis_index("s") * info.num_cores + lax.axis_index("c")
        for j in range(n_chunks):
            base = wid * b_per_w + j * chunk
            pltpu.sync_copy(idx_hbm.at[pl.ds(base, chunk)], idx_v)
            pltpu.async_copy(table_hbm.at[idx_v], rows_v, sem).wait()
            pltpu.sync_copy(rows_v, out_hbm.at[pl.ds(base, chunk)])

    return gather_kernel(table, idx)[:, :D]


def kernel(x, conv1_W, conv1_b, bn1_g, bn1_b, conv2_W, conv2_b,
           res_W3, res_b3, res_g1, res_be1, res_W1, res_b1, res_g2, res_be2,
           deconv1_W, deconv1_b, bn2_g, bn2_b, deconv2_W, deconv2_b, codebook):
    h = _conv(x, conv1_W, conv1_b, 2, 1)
    h = _bn(h, bn1_g, bn1_b)
    h = jax.nn.relu(h)
    h = _conv(h, conv2_W, conv2_b, 2, 1)
    h = _resblock(h, 0, res_W3, res_b3, res_g1, res_be1, res_W1, res_b1, res_g2, res_be2)
    z_e_x = _resblock(h, 1, res_W3, res_b3, res_g1, res_be1, res_W1, res_b1, res_g2, res_be2)
    z_e_perm = z_e_x.transpose(0, 2, 3, 1)
    flat = z_e_perm.reshape(-1, z_e_perm.shape[-1])
    idx = _vq_indices(flat, codebook)
    zq_flat = _sc_gather(codebook, idx)
    z_q_perm = zq_flat.reshape(z_e_perm.shape)
    z_q_x = z_q_perm.transpose(0, 3, 1, 2)
    z_q_x_st = z_e_x + lax.stop_gradient(z_q_x - z_e_x)
    h = _resblock(z_q_x_st, 2, res_W3, res_b3, res_g1, res_be1, res_W1, res_b1, res_g2, res_be2)
    h = _resblock(h, 3, res_W3, res_b3, res_g1, res_be1, res_W1, res_b1, res_g2, res_be2)
    h = jax.nn.relu(h)
    h = _deconv(h, deconv1_W, deconv1_b, 2, 1, 4)
    h = _bn(h, bn2_g, bn2_b)
    h = jax.nn.relu(h)
    h = _deconv(h, deconv2_W, deconv2_b, 2, 1, 4)
    x_tilde = jnp.tanh(h)
    return (x_tilde, z_e_x, z_q_x)
